# Initial kernel scaffold; baseline (speedup 1.0000x reference)
import functools

import jax
import jax.numpy as jnp
from jax import lax
from jax.experimental import pallas as pl
from jax.experimental.pallas import tpu as pltpu
from jax.experimental.pallas import tpu_sc as plsc

N = 10000
E = 320000
D = 128
EPS = 1e-5

NC = 2
NS = 16
NW = NC * NS

CH = 128

DEG_W = 16
NPAD = 10240
DEG_EPT = E // NS
DEG_NCH = DEG_EPT // CH
DEG_REM = DEG_EPT - DEG_NCH * CH
DEG_RPT = NPAD // NS

EPT = E // NW
ANCH = EPT // CH
AREM = EPT - ANCH * CH
RPT = N // NS
RCH = 125

_MESH = dict(core_axis_name="c", subcore_axis_name="s")


def _rsqrt16(x):
    bits = plsc.bitcast(x, jnp.int32)
    i = jnp.int32(0x5F3759DF) - lax.shift_right_logical(bits, 1)
    y = plsc.bitcast(i, jnp.float32)
    for _ in range(3):
        y = y * (jnp.float32(1.5) - jnp.float32(0.5) * x * y * y)
    return y


def _sc_norms(edge_index, zeros_deg, ones_rows):
    mesh = plsc.VectorSubcoreMesh(**_MESH)

    @functools.partial(
        pl.kernel,
        out_type=jax.ShapeDtypeStruct((NC, NPAD), jnp.float32),
        mesh=mesh,
        scratch_types=[
            pltpu.VMEM_SHARED((NPAD, DEG_W), jnp.float32),
            pltpu.VMEM((CH, DEG_W), jnp.float32),
            pltpu.VMEM((CH,), jnp.int32),
            pltpu.VMEM((DEG_REM,), jnp.int32),
            pltpu.VMEM((DEG_RPT, DEG_W), jnp.float32),
            pltpu.VMEM((DEG_RPT,), jnp.float32),
        ],
    )
    def kern(edge_hbm, zeros_hbm, ones_hbm, norms_hbm,
             acc, ones_b, idx_b, ridx_b, stripe_b, nbuf):
        cid = lax.axis_index("c")
        sid = lax.axis_index("s")
        pltpu.sync_copy(zeros_hbm, stripe_b)
        pltpu.sync_copy(stripe_b, acc.at[pl.ds(sid * DEG_RPT, DEG_RPT)])
        pltpu.sync_copy(ones_hbm, ones_b)
        plsc.subcore_barrier()

        base = sid * DEG_EPT

        def body(c, carry):
            off = base + c * CH
            pltpu.sync_copy(edge_hbm.at[cid, pl.ds(off, CH)], idx_b)
            pltpu.sync_copy(ones_b, acc.at[idx_b], add=True)
            return carry

        lax.fori_loop(0, DEG_NCH, body, 0)
        roff = base + DEG_NCH * CH
        pltpu.sync_copy(edge_hbm.at[cid, pl.ds(roff, DEG_REM)], ridx_b)
        pltpu.sync_copy(ones_b.at[pl.ds(0, DEG_REM)], acc.at[ridx_b], add=True)
        plsc.subcore_barrier()

        pltpu.sync_copy(acc.at[pl.ds(sid * DEG_RPT, DEG_RPT)], stripe_b)
        iota16 = lax.iota(jnp.int32, 16)
        zero16 = jnp.zeros((16,), jnp.int32)

        def nbody(j, carry):
            rows = j * 16 + iota16
            d = plsc.load_gather(stripe_b, [rows, zero16])
            r = _rsqrt16(jnp.maximum(d, jnp.float32(1.0)))
            nbuf[pl.ds(j * 16, 16)] = r
            return carry

        lax.fori_loop(0, DEG_RPT // 16, nbody, 0)
        pltpu.sync_copy(nbuf, norms_hbm.at[cid, pl.ds(sid * DEG_RPT, DEG_RPT)])

    return kern(edge_index, zeros_deg, ones_rows)


def _sc_aggregate(hn, edge_index, zeros_rows):
    mesh = plsc.VectorSubcoreMesh(**_MESH)

    @functools.partial(
        pl.kernel,
        out_type=jax.ShapeDtypeStruct((NC, N, D), jnp.float32),
        mesh=mesh,
        scratch_types=[
            pltpu.VMEM_SHARED((N, D), jnp.float32),
            pltpu.VMEM((CH,), jnp.int32),
            pltpu.VMEM((CH,), jnp.int32),
            pltpu.VMEM((CH, D), jnp.float32),
            pltpu.VMEM((AREM,), jnp.int32),
            pltpu.VMEM((AREM,), jnp.int32),
            pltpu.VMEM((AREM, D), jnp.float32),
            pltpu.SemaphoreType.DMA,
        ],
    )
    def kern(hn_hbm, edge_hbm, zeros_hbm, out_hbm,
             acc, sidx, didx, rows, rs, rd, rrows, sem):
        cid = lax.axis_index("c")
        sid = lax.axis_index("s")
        wid = cid * NS + sid
        pltpu.sync_copy(zeros_hbm, rows.at[pl.ds(0, RCH)])
        for k in range(RPT // RCH):
            pltpu.sync_copy(rows.at[pl.ds(0, RCH)],
                            acc.at[pl.ds(sid * RPT + k * RCH, RCH)])
        plsc.subcore_barrier()

        base = wid * EPT

        def body(c, carry):
            off = base + c * CH
            pltpu.sync_copy(edge_hbm.at[0, pl.ds(off, CH)], sidx)
            pltpu.sync_copy(edge_hbm.at[1, pl.ds(off, CH)], didx)
            pltpu.async_copy(hn_hbm.at[sidx], rows, sem).wait()
            pltpu.sync_copy(rows, acc.at[didx], add=True)
            return carry

        lax.fori_loop(0, ANCH, body, 0)
        roff = base + ANCH * CH
        pltpu.sync_copy(edge_hbm.at[0, pl.ds(roff, AREM)], rs)
        pltpu.sync_copy(edge_hbm.at[1, pl.ds(roff, AREM)], rd)
        pltpu.async_copy(hn_hbm.at[rs], rrows, sem).wait()
        pltpu.sync_copy(rrows, acc.at[rd], add=True)
        plsc.subcore_barrier()

        for k in range(RPT // RCH):
            r0 = sid * RPT + k * RCH
            pltpu.sync_copy(acc.at[pl.ds(r0, RCH)], rows.at[pl.ds(0, RCH)])
            pltpu.sync_copy(rows.at[pl.ds(0, RCH)], out_hbm.at[cid, pl.ds(r0, RCH)])

    return kern(hn, edge_index, zeros_rows)


def _tc_scale(h, ns):
    def body(h_ref, ns_ref, o_ref):
        o_ref[...] = h_ref[...] * ns_ref[...]

    return pl.pallas_call(
        body, out_shape=jax.ShapeDtypeStruct((N, D), jnp.float32)
    )(h, ns)


def _bn_relu(y, g, bt):
    mu = jnp.mean(y, axis=0, keepdims=True)
    yc = y - mu
    var = jnp.mean(yc * yc, axis=0, keepdims=True)
    z = g * (yc * lax.rsqrt(var + EPS)) + bt
    return jnp.maximum(z, 0.0)


def _tc_dense_mid(parts, nd, ns, W, b, g, bt):
    def body(p_ref, nd_ref, ns_ref, W_ref, b_ref, g_ref, bt_ref, o_ref):
        x = (p_ref[0] + p_ref[1]) * nd_ref[...]
        y = jnp.dot(x, W_ref[...], preferred_element_type=jnp.float32) + b_ref[...]
        z = _bn_relu(y, g_ref[...], bt_ref[...])
        o_ref[...] = z * ns_ref[...]

    return pl.pallas_call(
        body, out_shape=jax.ShapeDtypeStruct((N, D), jnp.float32)
    )(parts, nd, ns, W, b, g, bt)


def _tc_dense_last(parts, nd, W, b, g, bt, W_fc, b_fc):
    def body(p_ref, nd_ref, W_ref, b_ref, g_ref, bt_ref, Wf_ref, bf_ref, o_ref):
        x = (p_ref[0] + p_ref[1]) * nd_ref[...]
        y = jnp.dot(x, W_ref[...], preferred_element_type=jnp.float32) + b_ref[...]
        z = _bn_relu(y, g_ref[...], bt_ref[...])
        o_ref[...] = (
            jnp.dot(z, Wf_ref[...], preferred_element_type=jnp.float32) + bf_ref[...]
        )

    return pl.pallas_call(
        body, out_shape=jax.ShapeDtypeStruct((N, D), jnp.float32)
    )(parts, nd, W, b, g, bt, W_fc, b_fc)


def kernel(h, edge_index, W0, b0, gamma0, beta0, W1, b1, gamma1, beta1,
           W2, b2, gamma2, beta2, W_fc, b_fc):
    zeros_deg = jnp.zeros((DEG_RPT, DEG_W), jnp.float32)
    ones_rows = jnp.ones((CH, DEG_W), jnp.float32)
    zeros_rows = jnp.zeros((RCH, D), jnp.float32)

    norms = _sc_norms(edge_index, zeros_deg, ones_rows)
    ns = norms[0, :N].reshape(N, 1)
    nd = norms[1, :N].reshape(N, 1)

    hn = _tc_scale(h, ns)
    for W, b, g, bt in [(W0, b0, gamma0, beta0), (W1, b1, gamma1, beta1)]:
        parts = _sc_aggregate(hn, edge_index, zeros_rows)
        hn = _tc_dense_mid(parts, nd, ns, W, b.reshape(1, D), g.reshape(1, D),
                           bt.reshape(1, D))
    parts = _sc_aggregate(hn, edge_index, zeros_rows)
    out = _tc_dense_last(parts, nd, W2, b2.reshape(1, D), gamma2.reshape(1, D),
                         beta2.reshape(1, D), W_fc, b_fc.reshape(1, D))
    return out


# speedup vs baseline: 6.1522x; 6.1522x over previous
import functools

import jax
import jax.numpy as jnp
from jax import lax
from jax.experimental import pallas as pl
from jax.experimental.pallas import tpu as pltpu
from jax.experimental.pallas import tpu_sc as plsc

N = 10000
E = 320000
D = 128
EPS = 1e-5

NC = 2
NS = 16
NW = NC * NS

CH = 128

DEG_W = 16
NPAD = 10240
DEG_EPT = E // NS
DEG_NCH = DEG_EPT // CH
DEG_REM = DEG_EPT - DEG_NCH * CH
DEG_RPT = NPAD // NS

EPT = E // NW
ANCH = EPT // CH
AREM = EPT - ANCH * CH
RPT = NPAD // NS
RNCH = RPT // CH

_MESH = dict(core_axis_name="c", subcore_axis_name="s")


def _rsqrt16(x):
    bits = lax.bitcast_convert_type(x, jnp.int32)
    i = jnp.int32(0x5F3759DF) - lax.shift_right_logical(bits, 1)
    y = lax.bitcast_convert_type(i, jnp.float32)
    for _ in range(3):
        y = y * (jnp.float32(1.5) - jnp.float32(0.5) * x * y * y)
    return y


def _sc_norms(edge_index, zeros_deg, ones_rows):
    mesh = plsc.VectorSubcoreMesh(**_MESH)

    @functools.partial(
        pl.kernel,
        out_type=jax.ShapeDtypeStruct((NC, NPAD), jnp.float32),
        mesh=mesh,
        scratch_types=[
            pltpu.VMEM_SHARED((NPAD,), jnp.float32),
            pltpu.VMEM((CH,), jnp.float32),
            pltpu.VMEM((CH,), jnp.int32),
            pltpu.VMEM((DEG_REM,), jnp.int32),
            pltpu.VMEM((DEG_RPT,), jnp.float32),
            pltpu.VMEM((DEG_RPT,), jnp.float32),
        ],
    )
    def kern(edge_hbm, zeros_hbm, ones_hbm, norms_hbm,
             acc, ones_b, idx_b, ridx_b, stripe_b, nbuf):
        cid = lax.axis_index("c")
        sid = lax.axis_index("s")
        pltpu.sync_copy(zeros_hbm, stripe_b)
        pltpu.sync_copy(stripe_b, acc.at[pl.ds(sid * DEG_RPT, DEG_RPT)])
        pltpu.sync_copy(ones_hbm, ones_b)
        plsc.subcore_barrier()

        base = sid * DEG_EPT

        ebase = cid * E + base

        def body(c, carry):
            off = ebase + c * CH
            pltpu.sync_copy(edge_hbm.at[pl.ds(off, CH)], idx_b)
            pltpu.sync_copy(ones_b, acc.at[idx_b], add=True)
            return carry

        lax.fori_loop(0, DEG_NCH, body, 0)
        roff = ebase + DEG_NCH * CH
        pltpu.sync_copy(edge_hbm.at[pl.ds(roff, DEG_REM)], ridx_b)
        pltpu.sync_copy(ones_b.at[pl.ds(0, DEG_REM)], acc.at[ridx_b], add=True)
        plsc.subcore_barrier()

        pltpu.sync_copy(acc.at[pl.ds(sid * DEG_RPT, DEG_RPT)], stripe_b)

        def nbody(j, carry):
            d = stripe_b[pl.ds(j * 16, 16)]
            nbuf[pl.ds(j * 16, 16)] = _rsqrt16(jnp.maximum(d, jnp.float32(1.0)))
            return carry

        lax.fori_loop(0, DEG_RPT // 16, nbody, 0)
        pltpu.sync_copy(nbuf, norms_hbm.at[cid, pl.ds(sid * DEG_RPT, DEG_RPT)])

    return kern(edge_index, zeros_deg, ones_rows)


def _sc_aggregate(hn, edge_index, zeros_rows):
    mesh = plsc.VectorSubcoreMesh(**_MESH)

    @functools.partial(
        pl.kernel,
        out_type=jax.ShapeDtypeStruct((NC, NPAD, D), jnp.float32),
        mesh=mesh,
        scratch_types=[
            pltpu.VMEM_SHARED((NPAD, D), jnp.float32),
            pltpu.VMEM((CH,), jnp.int32),
            pltpu.VMEM((CH,), jnp.int32),
            pltpu.VMEM((CH, D), jnp.float32),
            pltpu.VMEM((AREM,), jnp.int32),
            pltpu.VMEM((AREM,), jnp.int32),
            pltpu.VMEM((AREM, D), jnp.float32),
            pltpu.SemaphoreType.DMA,
        ],
    )
    def kern(hn_hbm, edge_hbm, zeros_hbm, out_hbm,
             acc, sidx, didx, rows, rs, rd, rrows, sem):
        cid = lax.axis_index("c")
        sid = lax.axis_index("s")
        wid = cid * NS + sid
        pltpu.sync_copy(zeros_hbm, rows)
        for k in range(RNCH):
            pltpu.sync_copy(rows, acc.at[pl.ds(sid * RPT + k * CH, CH)])
        plsc.subcore_barrier()

        base = wid * EPT

        def body(c, carry):
            off = base + c * CH
            pltpu.sync_copy(edge_hbm.at[pl.ds(off, CH)], sidx)
            pltpu.sync_copy(edge_hbm.at[pl.ds(E + off, CH)], didx)
            pltpu.async_copy(hn_hbm.at[sidx], rows, sem).wait()
            pltpu.sync_copy(rows, acc.at[didx], add=True)
            return carry

        lax.fori_loop(0, ANCH, body, 0)
        roff = base + ANCH * CH
        pltpu.sync_copy(edge_hbm.at[pl.ds(roff, AREM)], rs)
        pltpu.sync_copy(edge_hbm.at[pl.ds(E + roff, AREM)], rd)
        pltpu.async_copy(hn_hbm.at[rs], rrows, sem).wait()
        pltpu.sync_copy(rrows, acc.at[rd], add=True)
        plsc.subcore_barrier()

        for k in range(RNCH):
            r0 = sid * RPT + k * CH
            pltpu.sync_copy(acc.at[pl.ds(r0, CH)], rows)
            pltpu.sync_copy(rows, out_hbm.at[cid, pl.ds(r0, CH)])

    return kern(hn, edge_index, zeros_rows)


def _tc_scale(h, ns):
    def body(h_ref, ns_ref, o_ref):
        o_ref[...] = h_ref[...] * ns_ref[...]

    return pl.pallas_call(
        body, out_shape=jax.ShapeDtypeStruct((N, D), jnp.float32)
    )(h, ns)


def _bn_relu(y, g, bt):
    mu = jnp.mean(y, axis=0, keepdims=True)
    yc = y - mu
    var = jnp.mean(yc * yc, axis=0, keepdims=True)
    z = g * (yc * lax.rsqrt(var + EPS)) + bt
    return jnp.maximum(z, 0.0)


def _tc_dense_mid(parts, nd, ns, W, b, g, bt):
    def body(p_ref, nd_ref, ns_ref, W_ref, b_ref, g_ref, bt_ref, o_ref):
        x = (p_ref[0, :N] + p_ref[1, :N]) * nd_ref[...]
        y = jnp.dot(x, W_ref[...], preferred_element_type=jnp.float32) + b_ref[...]
        z = _bn_relu(y, g_ref[...], bt_ref[...])
        o_ref[...] = z * ns_ref[...]

    return pl.pallas_call(
        body, out_shape=jax.ShapeDtypeStruct((N, D), jnp.float32)
    )(parts, nd, ns, W, b, g, bt)


def _tc_dense_last(parts, nd, W, b, g, bt, W_fc, b_fc):
    def body(p_ref, nd_ref, W_ref, b_ref, g_ref, bt_ref, Wf_ref, bf_ref, o_ref):
        x = (p_ref[0, :N] + p_ref[1, :N]) * nd_ref[...]
        y = jnp.dot(x, W_ref[...], preferred_element_type=jnp.float32) + b_ref[...]
        z = _bn_relu(y, g_ref[...], bt_ref[...])
        o_ref[...] = (
            jnp.dot(z, Wf_ref[...], preferred_element_type=jnp.float32) + bf_ref[...]
        )

    return pl.pallas_call(
        body, out_shape=jax.ShapeDtypeStruct((N, D), jnp.float32)
    )(parts, nd, W, b, g, bt, W_fc, b_fc)


def kernel(h, edge_index, W0, b0, gamma0, beta0, W1, b1, gamma1, beta1,
           W2, b2, gamma2, beta2, W_fc, b_fc):
    zeros_deg = jnp.zeros((DEG_RPT,), jnp.float32)
    ones_rows = jnp.ones((CH,), jnp.float32)
    zeros_rows = jnp.zeros((CH, D), jnp.float32)

    eidx = edge_index.reshape(2 * E)
    norms = _sc_norms(eidx, zeros_deg, ones_rows)
    ns = norms[0, :N].reshape(N, 1)
    nd = norms[1, :N].reshape(N, 1)

    hn = _tc_scale(h, ns)
    for W, b, g, bt in [(W0, b0, gamma0, beta0), (W1, b1, gamma1, beta1)]:
        parts = _sc_aggregate(hn, eidx, zeros_rows)
        hn = _tc_dense_mid(parts, nd, ns, W, b.reshape(1, D), g.reshape(1, D),
                           bt.reshape(1, D))
    parts = _sc_aggregate(hn, eidx, zeros_rows)
    out = _tc_dense_last(parts, nd, W2, b2.reshape(1, D), gamma2.reshape(1, D),
                         beta2.reshape(1, D), W_fc, b_fc.reshape(1, D))
    return out
